# baseline (speedup 1.0000x reference)
import functools

import jax
import jax.numpy as jnp
from jax import lax
from jax.experimental import pallas as pl
from jax.experimental.pallas import tpu as pltpu
from jax.experimental.pallas import tpu_sc as plsc

B, T, D = 4096, 16, 768
R = B * T
NC, NS = 2, 16
NW = NC * NS
ROWS_PER_W = R // NW
CHUNK = 64
NCHUNK = ROWS_PER_W // CHUNK
NV = D // 16
EPS = 1e-12


def _rsqrt(x):
    i = lax.bitcast_convert_type(x, jnp.int32)
    y = lax.bitcast_convert_type(jnp.int32(0x5F3759DF) - (i >> 1), jnp.float32)
    for _ in range(3):
        y = y * (1.5 - 0.5 * x * y * y)
    return y


def _lane_sum(v):
    lane = lax.iota(jnp.int32, 16)
    for d in (1, 2, 4, 8):
        v = v + v.at[lane ^ d].get(mode="promise_in_bounds")
    return v


@functools.partial(
    pl.kernel,
    mesh=plsc.VectorSubcoreMesh(core_axis_name="c", subcore_axis_name="s"),
    out_type=jax.ShapeDtypeStruct((R, D), jnp.float32),
    scratch_types=[
        pltpu.VMEM((CHUNK, D), jnp.float32),
        pltpu.VMEM((T, D), jnp.float32),
        pltpu.VMEM((D,), jnp.float32),
        pltpu.VMEM((D,), jnp.float32),
    ],
)
def _ln_kernel(emb, pos, gamma, beta, out, buf, pos_v, g_v, b_v):
    wid = lax.axis_index("s") * NC + lax.axis_index("c")
    base = wid * ROWS_PER_W
    pltpu.sync_copy(pos, pos_v)
    pltpu.sync_copy(gamma, g_v)
    pltpu.sync_copy(beta, b_v)

    def chunk_body(ci, _):
        row0 = base + ci * CHUNK
        pltpu.sync_copy(emb.at[pl.ds(row0, CHUNK)], buf)

        def row_body(r, _):
            t = lax.rem(r, T)
            s = jnp.zeros((16,), jnp.float32)
            s2 = jnp.zeros((16,), jnp.float32)
            for j in range(NV):
                v = buf[r, pl.ds(j * 16, 16)] + pos_v[t, pl.ds(j * 16, 16)]
                buf[r, pl.ds(j * 16, 16)] = v
                s = s + v
                s2 = s2 + v * v
            mean = _lane_sum(s) * (1.0 / D)
            var = _lane_sum(s2) * (1.0 / D) - mean * mean
            rs = _rsqrt(var + EPS)
            for j in range(NV):
                v = buf[r, pl.ds(j * 16, 16)]
                buf[r, pl.ds(j * 16, 16)] = (
                    (v - mean) * rs * g_v[pl.ds(j * 16, 16)]
                    + b_v[pl.ds(j * 16, 16)]
                )
            return 0

        lax.fori_loop(0, CHUNK, row_body, 0)
        pltpu.sync_copy(buf, out.at[pl.ds(row0, CHUNK)])
        return 0

    lax.fori_loop(0, NCHUNK, chunk_body, 0)


def kernel(embeddings, pos_table, ln_gamma, ln_beta):
    out = _ln_kernel(embeddings.reshape(R, D), pos_table, ln_gamma, ln_beta)
    return out.reshape(B, T, D)


# speedup vs baseline: 1.9675x; 1.9675x over previous
import functools

import jax
import jax.numpy as jnp
from jax import lax
from jax.experimental import pallas as pl
from jax.experimental.pallas import tpu as pltpu
from jax.experimental.pallas import tpu_sc as plsc

B, T, D = 4096, 16, 768
R = B * T
NC, NS = 2, 16
NW = NC * NS
ROWS_PER_W = R // NW
CHUNK = 64
NCHUNK = ROWS_PER_W // CHUNK
NV = D // 16
EPS = 1e-12


def _rsqrt(x):
    i = lax.bitcast_convert_type(x, jnp.int32)
    y = lax.bitcast_convert_type(jnp.int32(0x5F3759DF) - (i >> 1), jnp.float32)
    for _ in range(3):
        y = y * (1.5 - 0.5 * x * y * y)
    return y


def _lane_sum(v):
    lane = lax.iota(jnp.int32, 16)
    for d in (1, 2, 4, 8):
        v = v + v.at[lane ^ d].get(mode="promise_in_bounds")
    return v


@functools.partial(
    pl.kernel,
    mesh=plsc.VectorSubcoreMesh(core_axis_name="c", subcore_axis_name="s"),
    out_type=jax.ShapeDtypeStruct((R, D), jnp.float32),
    scratch_types=[
        pltpu.VMEM((CHUNK, D), jnp.float32),
        pltpu.VMEM((T, D), jnp.float32),
        pltpu.VMEM((D,), jnp.float32),
        pltpu.VMEM((D,), jnp.float32),
    ],
)
def _ln_kernel(emb, pos, gamma, beta, out, buf, pos_v, g_v, b_v):
    wid = lax.axis_index("s") * NC + lax.axis_index("c")
    base = wid * ROWS_PER_W
    pltpu.sync_copy(pos, pos_v)
    pltpu.sync_copy(gamma, g_v)
    pltpu.sync_copy(beta, b_v)

    G = CHUNK // T

    def chunk_body(ci, _):
        row0 = base + ci * CHUNK
        pltpu.sync_copy(emb.at[pl.ds(row0, CHUNK)], buf)

        def group_body(t, _):
            rows = [t + T * i for i in range(G)]
            s = [jnp.zeros((16,), jnp.float32) for _ in range(G)]
            s2 = [jnp.zeros((16,), jnp.float32) for _ in range(G)]
            for j in range(NV):
                js = pl.ds(j * 16, 16)
                p = pos_v[t, js]
                for i in range(G):
                    v = buf[rows[i], js] + p
                    buf[rows[i], js] = v
                    s[i] = s[i] + v
                    s2[i] = s2[i] + v * v
            mean = [_lane_sum(s[i]) * (1.0 / D) for i in range(G)]
            var = [
                _lane_sum(s2[i]) * (1.0 / D) - mean[i] * mean[i]
                for i in range(G)
            ]
            rs = [_rsqrt(var[i] + EPS) for i in range(G)]
            for j in range(NV):
                js = pl.ds(j * 16, 16)
                gj = g_v[js]
                bj = b_v[js]
                for i in range(G):
                    v = buf[rows[i], js]
                    buf[rows[i], js] = (v - mean[i]) * rs[i] * gj + bj
            return 0

        lax.fori_loop(0, T, group_body, 0)
        pltpu.sync_copy(buf, out.at[pl.ds(row0, CHUNK)])
        return 0

    lax.fori_loop(0, NCHUNK, chunk_body, 0)


def kernel(embeddings, pos_table, ln_gamma, ln_beta):
    out = _ln_kernel(embeddings.reshape(R, D), pos_table, ln_gamma, ln_beta)
    return out.reshape(B, T, D)


# speedup vs baseline: 1.9688x; 1.0006x over previous
import functools

import jax
import jax.numpy as jnp
from jax import lax
from jax.experimental import pallas as pl
from jax.experimental.pallas import tpu as pltpu
from jax.experimental.pallas import tpu_sc as plsc

B, T, D = 4096, 16, 768
R = B * T
NC, NS = 2, 16
NW = NC * NS
ROWS_PER_W = R // NW
CHUNK = 64
NCHUNK = ROWS_PER_W // CHUNK
NV = D // 16
EPS = 1e-12


def _rsqrt(x):
    i = lax.bitcast_convert_type(x, jnp.int32)
    y = lax.bitcast_convert_type(jnp.int32(0x5F3759DF) - (i >> 1), jnp.float32)
    for _ in range(3):
        y = y * (1.5 - 0.5 * x * y * y)
    return y


def _lane_sum(v):
    lane = lax.iota(jnp.int32, 16)
    for d in (1, 2, 4, 8):
        v = v + v.at[lane ^ d].get(mode="promise_in_bounds")
    return v


@functools.partial(
    pl.kernel,
    mesh=plsc.VectorSubcoreMesh(core_axis_name="c", subcore_axis_name="s"),
    out_type=jax.ShapeDtypeStruct((R, D), jnp.float32),
    scratch_types=[
        pltpu.VMEM((CHUNK, D), jnp.float32),
        pltpu.VMEM((T, D), jnp.float32),
        pltpu.VMEM((D,), jnp.float32),
        pltpu.VMEM((D,), jnp.float32),
    ],
)
def _ln_kernel(emb, pos, gamma, beta, out, buf, pos_v, g_v, b_v):
    wid = lax.axis_index("s") * NC + lax.axis_index("c")
    base = wid * ROWS_PER_W
    pltpu.sync_copy(pos, pos_v)
    pltpu.sync_copy(gamma, g_v)
    pltpu.sync_copy(beta, b_v)

    G = CHUNK // T

    def chunk_body(ci, _):
        row0 = base + ci * CHUNK
        pltpu.sync_copy(emb.at[pl.ds(row0, CHUNK)], buf)

        @plsc.parallel_loop(0, T)
        def group_body(t):
            rows = [t + T * i for i in range(G)]
            s = [jnp.zeros((16,), jnp.float32) for _ in range(G)]
            s2 = [jnp.zeros((16,), jnp.float32) for _ in range(G)]
            for j in range(NV):
                js = pl.ds(j * 16, 16)
                p = pos_v[t, js]
                for i in range(G):
                    v = buf[rows[i], js] + p
                    buf[rows[i], js] = v
                    s[i] = s[i] + v
                    s2[i] = s2[i] + v * v
            mean = [_lane_sum(s[i]) * (1.0 / D) for i in range(G)]
            var = [
                _lane_sum(s2[i]) * (1.0 / D) - mean[i] * mean[i]
                for i in range(G)
            ]
            rs = [_rsqrt(var[i] + EPS) for i in range(G)]
            for j in range(NV):
                js = pl.ds(j * 16, 16)
                gj = g_v[js]
                bj = b_v[js]
                for i in range(G):
                    v = buf[rows[i], js]
                    buf[rows[i], js] = (v - mean[i]) * rs[i] * gj + bj

        pltpu.sync_copy(buf, out.at[pl.ds(row0, CHUNK)])
        return 0

    lax.fori_loop(0, NCHUNK, chunk_body, 0)


def kernel(embeddings, pos_table, ln_gamma, ln_beta):
    out = _ln_kernel(embeddings.reshape(R, D), pos_table, ln_gamma, ln_beta)
    return out.reshape(B, T, D)
